# Initial kernel scaffold; baseline (speedup 1.0000x reference)
import jax
import jax.numpy as jnp
from jax.experimental import pallas as pl


def kernel(inputs, batch_size, embedding, attention_1, attention_2, attention_3):
    raise NotImplementedError("write your pallas kernel here")


# speedup vs baseline: 2.2904x; 2.2904x over previous
import functools

import jax
import jax.numpy as jnp
from jax import lax
from jax.experimental import pallas as pl
from jax.experimental.pallas import tpu as pltpu
from jax.experimental.pallas import tpu_sc as plsc

HIDDEN = 64
BATCH = 16384
SEQ = 50

NC = 2
NS = 16
NW = NC * NS
BPW = BATCH // NW
CB = 16
ROWS = CB * SEQ
G = 80
NG = ROWS // G
NCHUNK = BPW // CB

_mesh = plsc.VectorSubcoreMesh(core_axis_name="c", subcore_axis_name="s")


@functools.partial(
    pl.kernel,
    mesh=_mesh,
    compiler_params=pltpu.CompilerParams(use_tc_tiling_on_sc=False),
    out_type=jax.ShapeDtypeStruct((BATCH * 3 * HIDDEN,), jnp.float32),
    scratch_types=[
        pltpu.VMEM((ROWS,), jnp.int32),
        pltpu.VMEM((ROWS, HIDDEN), jnp.float32),
        pltpu.VMEM((CB * 3 * HIDDEN,), jnp.float32),
        pltpu.VMEM((3 * HIDDEN,), jnp.float32),
        pltpu.SemaphoreType.DMA,
    ],
)
def _sc_encoder(idx_hbm, table_hbm, att_hbm, out_hbm,
                idx_v, rows_v, out_v, att_v, gsem):
    wid = lax.axis_index("c") * NS + lax.axis_index("s")
    pltpu.sync_copy(att_hbm, att_v)
    atts = [att_v[pl.ds(k * 16, 16)] for k in range(12)]
    lanes = lax.iota(jnp.int32, 16)
    perms = [(lanes ^ s)[:, None] for s in (8, 4, 2, 1)]
    _dnums = lax.GatherDimensionNumbers(
        offset_dims=(), collapsed_slice_dims=(0,), start_index_map=(0,))

    def lane_sum(v):
        for p in perms:
            v = v + lax.gather(v, p, _dnums, slice_sizes=(1,),
                               mode=lax.GatherScatterMode.PROMISE_IN_BOUNDS)
        return v

    def chunk_body(c, carry):
        row_base = wid * (BPW * SEQ) + c * ROWS
        pltpu.sync_copy(idx_hbm.at[pl.ds(row_base, ROWS)], idx_v)
        copies = []
        for g in range(NG):
            cp = pltpu.make_async_copy(
                table_hbm.at[idx_v.at[pl.ds(g * G, G)]],
                rows_v.at[pl.ds(g * G, G)],
                gsem,
            )
            cp.start()
            copies.append(cp)
        for cp in copies:
            cp.wait()

        def b_body(b, bcarry):
            def row_body(l, acc):
                r = b * SEQ + l
                e = [rows_v[r, pl.ds(j * 16, 16)] for j in range(4)]
                new = list(acc)
                for i in range(3):
                    m = (e[0] * atts[4 * i] + e[1] * atts[4 * i + 1]
                         + e[2] * atts[4 * i + 2] + e[3] * atts[4 * i + 3])
                    alpha = lane_sum(m)
                    for j in range(4):
                        new[4 * i + j] = new[4 * i + j] + alpha * e[j]
                return tuple(new)

            acc0 = tuple(jnp.zeros((16,), jnp.float32) for _ in range(12))
            acc = lax.fori_loop(0, SEQ, row_body, acc0)
            ob = pl.multiple_of(b * (3 * HIDDEN), 3 * HIDDEN)
            for k in range(12):
                out_v[pl.ds(ob + k * 16, 16)] = acc[k]
            return bcarry

        lax.fori_loop(0, CB, b_body, 0)
        out_base = (wid * BPW + c * CB) * (3 * HIDDEN)
        pltpu.sync_copy(out_v, out_hbm.at[pl.ds(out_base, CB * 3 * HIDDEN)])
        return carry

    lax.fori_loop(0, NCHUNK, chunk_body, 0)


def kernel(inputs, batch_size, embedding, attention_1, attention_2, attention_3):
    idx = inputs.reshape(-1).astype(jnp.int32)
    att = jnp.concatenate(
        [attention_1.reshape(-1), attention_2.reshape(-1),
         attention_3.reshape(-1)], axis=0)
    out = _sc_encoder(idx, embedding, att)
    return out.reshape(BATCH, 3, HIDDEN)


# speedup vs baseline: 2.4544x; 1.0716x over previous
import functools

import jax
import jax.numpy as jnp
from jax import lax
from jax.experimental import pallas as pl
from jax.experimental.pallas import tpu as pltpu
from jax.experimental.pallas import tpu_sc as plsc

HIDDEN = 64
BATCH = 16384
SEQ = 50

NC = 2
NS = 16
NW = NC * NS
BPW = BATCH // NW
CB = 16
ROWS = CB * SEQ
G = 80
NG = ROWS // G
NCHUNK = BPW // CB
OUTW = 3 * HIDDEN

_mesh = plsc.VectorSubcoreMesh(core_axis_name="c", subcore_axis_name="s")


@functools.partial(
    pl.kernel,
    mesh=_mesh,
    compiler_params=pltpu.CompilerParams(use_tc_tiling_on_sc=False),
    out_type=jax.ShapeDtypeStruct((BATCH * OUTW,), jnp.float32),
    scratch_types=[
        pltpu.VMEM((2, ROWS), jnp.int32),
        pltpu.VMEM((2, ROWS, HIDDEN), jnp.float32),
        pltpu.VMEM((2, CB * OUTW), jnp.float32),
        pltpu.VMEM((OUTW,), jnp.float32),
        pltpu.SemaphoreType.DMA,
        pltpu.SemaphoreType.DMA,
        pltpu.SemaphoreType.DMA,
    ],
)
def _sc_encoder(idx_hbm, table_hbm, att_hbm, out_hbm,
                idx_v, rows_v, out_v, att_v, sem0, sem1, osem):
    wid = lax.axis_index("c") * NS + lax.axis_index("s")
    pltpu.sync_copy(att_hbm, att_v)
    atts = [att_v[pl.ds(k * 16, 16)] for k in range(12)]
    lanes = lax.iota(jnp.int32, 16)
    perms = [(lanes ^ s)[:, None] for s in (8, 4, 2, 1)]
    _dnums = lax.GatherDimensionNumbers(
        offset_dims=(), collapsed_slice_dims=(0,), start_index_map=(0,))

    def lane_sum(v):
        for p in perms:
            v = v + lax.gather(v, p, _dnums, slice_sizes=(1,),
                               mode=lax.GatherScatterMode.PROMISE_IN_BOUNDS)
        return v

    sems = (sem0, sem1)

    def fire(c, s):
        row_base = pl.multiple_of(wid * (BPW * SEQ) + c * ROWS, ROWS)
        pltpu.sync_copy(idx_hbm.at[pl.ds(row_base, ROWS)], idx_v.at[s])
        for g in range(NG):
            pltpu.make_async_copy(
                table_hbm.at[idx_v.at[s].at[pl.ds(g * G, G)]],
                rows_v.at[s].at[pl.ds(g * G, G)],
                sems[s],
            ).start()

    def drain(s):
        for g in range(NG):
            pltpu.make_async_copy(
                table_hbm.at[idx_v.at[s].at[pl.ds(g * G, G)]],
                rows_v.at[s].at[pl.ds(g * G, G)],
                sems[s],
            ).wait()

    def write_out(c, s):
        out_base = pl.multiple_of((wid * BPW + c * CB) * OUTW, CB * OUTW)
        pltpu.make_async_copy(
            out_v.at[s], out_hbm.at[pl.ds(out_base, CB * OUTW)], osem).start()

    def drain_out(s):
        pltpu.make_async_copy(
            out_v.at[s], out_hbm.at[pl.ds(0, CB * OUTW)], osem).wait()

    def compute(s):
        def b_body(b, bcarry):
            def row_body(l2, acc):
                new = list(acc)
                for u in range(2):
                    r = b * SEQ + l2 * 2 + u
                    e = [rows_v[s, r, pl.ds(j * 16, 16)] for j in range(4)]
                    for i in range(3):
                        m = (e[0] * atts[4 * i] + e[1] * atts[4 * i + 1]
                             + e[2] * atts[4 * i + 2] + e[3] * atts[4 * i + 3])
                        alpha = lane_sum(m)
                        for j in range(4):
                            new[4 * i + j] = new[4 * i + j] + alpha * e[j]
                return tuple(new)

            acc0 = tuple(jnp.zeros((16,), jnp.float32) for _ in range(12))
            acc = lax.fori_loop(0, SEQ // 2, row_body, acc0)
            ob = pl.multiple_of(b * OUTW, OUTW)
            for k in range(12):
                out_v[s, pl.ds(ob + k * 16, 16)] = acc[k]
            return bcarry

        lax.fori_loop(0, CB, b_body, 0)

    fire(0, 0)

    def pair_body(c2, carry):
        for par in (0, 1):
            c = c2 * 2 + par

            @pl.when(c + 1 < NCHUNK)
            def _():
                fire(c + 1, 1 - par)

            drain(par)

            @pl.when(c >= 2)
            def _():
                drain_out(par)

            compute(par)
            write_out(c, par)
        return carry

    lax.fori_loop(0, NCHUNK // 2, pair_body, 0)
    drain_out(0)
    drain_out(1)


def kernel(inputs, batch_size, embedding, attention_1, attention_2, attention_3):
    idx = inputs.reshape(-1).astype(jnp.int32)
    att = jnp.concatenate(
        [attention_1.reshape(-1), attention_2.reshape(-1),
         attention_3.reshape(-1)], axis=0)
    out = _sc_encoder(idx, embedding, att)
    return out.reshape(BATCH, 3, HIDDEN)
